# Initial kernel scaffold; baseline (speedup 1.0000x reference)
import jax
import jax.numpy as jnp
from jax import lax
from jax.experimental import pallas as pl
from jax.experimental.pallas import tpu as pltpu
from jax.experimental.pallas import tpu_sc as plsc

B = 16384
L = 200
D = 32
HIDDEN = 256
NC, NS = 2, 16
NW = NC * NS
SPW = B // NW
CS = 4
NCH = SPW // CS
IPR = 100
RPS = L // IPR
NDMA = CS * RPS
ROWS = CS * L
UR = 8


def _emb_body(x_hbm, tbl_hbm, out_hbm, idx0, idx1, rows0, rows1, outv,
              sem0, sem1):
    wid = lax.axis_index("s") * NC + lax.axis_index("c")
    xrow0 = wid * (SPW * RPS)
    obase = wid * SPW

    def start(g, idxv, rowsv, sem):
        pltpu.sync_copy(x_hbm.at[pl.ds(xrow0 + g * NDMA, NDMA)], idxv)
        for j in range(NDMA):
            pltpu.async_copy(tbl_hbm.at[idxv.at[j]],
                             rowsv.at[pl.ds(j * IPR, IPR)], sem)

    def finish(g, idxv, rowsv, sem):
        for j in range(NDMA):
            pltpu.make_async_copy(tbl_hbm.at[idxv.at[j]],
                                  rowsv.at[pl.ds(j * IPR, IPR)], sem).wait()
        for s in range(CS):
            def body(i, accs, s=s):
                a0, a1 = accs
                r = s * L + i * UR
                for u in range(UR):
                    a0 = a0 + rowsv[r + u, pl.ds(0, 16)]
                    a1 = a1 + rowsv[r + u, pl.ds(16, 16)]
                return (a0, a1)
            z = jnp.zeros((16,), jnp.float32)
            a0, a1 = lax.fori_loop(0, L // UR, body, (z, z))
            outv[s, pl.ds(0, 16)] = a0
            outv[s, pl.ds(16, 16)] = a1
        pltpu.sync_copy(outv, out_hbm.at[pl.ds(obase + g * CS, CS)])

    start(0, idx0, rows0, sem0)
    start(1, idx1, rows1, sem1)

    def loop_body(t, carry):
        g = 2 * t
        finish(g, idx0, rows0, sem0)
        start(g + 2, idx0, rows0, sem0)
        finish(g + 1, idx1, rows1, sem1)
        start(g + 3, idx1, rows1, sem1)
        return carry

    lax.fori_loop(0, (NCH - 2) // 2, loop_body, 0)
    finish(NCH - 2, idx0, rows0, sem0)
    finish(NCH - 1, idx1, rows1, sem1)


def _embed_sum(x, weight):
    xr = x.reshape(B * RPS, IPR).astype(jnp.int32)
    f = pl.kernel(
        _emb_body,
        out_type=jax.ShapeDtypeStruct((B, D), jnp.float32),
        mesh=plsc.VectorSubcoreMesh(core_axis_name="c", subcore_axis_name="s"),
        scratch_types=[
            pltpu.VMEM((NDMA, IPR), jnp.int32),
            pltpu.VMEM((NDMA, IPR), jnp.int32),
            pltpu.VMEM((ROWS, D), jnp.float32),
            pltpu.VMEM((ROWS, D), jnp.float32),
            pltpu.VMEM((CS, D), jnp.float32),
            pltpu.SemaphoreType.DMA,
            pltpu.SemaphoreType.DMA,
        ],
    )
    return f(xr, weight)


def _mlp_body(s_ref, w1t_ref, b1_ref, w2t_ref, b2_ref, o_ref):
    h = jnp.dot(s_ref[...], w1t_ref[...], preferred_element_type=jnp.float32)
    h = jnp.maximum(h + b1_ref[...], 0.0)
    o_ref[...] = (jnp.dot(h, w2t_ref[...], preferred_element_type=jnp.float32)
                  + b2_ref[...])


def _mlp(s, W1, b1, W2, b2):
    BM = 2048
    return pl.pallas_call(
        _mlp_body,
        grid=(B // BM,),
        in_specs=[
            pl.BlockSpec((BM, D), lambda i: (i, 0)),
            pl.BlockSpec((D, HIDDEN), lambda i: (0, 0)),
            pl.BlockSpec((1, HIDDEN), lambda i: (0, 0)),
            pl.BlockSpec((HIDDEN, 1), lambda i: (0, 0)),
            pl.BlockSpec((1, 1), lambda i: (0, 0)),
        ],
        out_specs=pl.BlockSpec((BM, 1), lambda i: (i, 0)),
        out_shape=jax.ShapeDtypeStruct((B, 1), jnp.float32),
    )(s, W1.T, b1.reshape(1, HIDDEN), W2.T, b2.reshape(1, 1))


def kernel(x, weight, W1, b1, W2, b2):
    s = _embed_sum(x, weight)
    return _mlp(s, W1, b1, W2, b2)


# speedup vs baseline: 15.0823x; 15.0823x over previous
import jax
import jax.numpy as jnp
from jax import lax
from jax.experimental import pallas as pl
from jax.experimental.pallas import tpu as pltpu
from jax.experimental.pallas import tpu_sc as plsc

B = 16384
L = 200
D = 32
HIDDEN = 256
NC, NS = 2, 16
NW = NC * NS
SPW = B // NW
CS = 4
NCH = SPW // CS
IPR = 100
RPS = L // IPR
NDMA = CS * RPS
ROWS = CS * L
UR = 8


def _emb_body(x_hbm, tbl_hbm, out_hbm, idx0, idx1, rows0, rows1, outv,
              sem0, sem1):
    wid = lax.axis_index("s") * NC + lax.axis_index("c")
    xrow0 = wid * (SPW * RPS)
    obase = wid * SPW

    def start(g, idxv, rowsv, sem):
        pltpu.sync_copy(x_hbm.at[pl.ds(xrow0 + g * NDMA, NDMA)], idxv)
        for j in range(NDMA):
            pltpu.async_copy(tbl_hbm.at[idxv.at[j]],
                             rowsv.at[pl.ds(j * IPR, IPR)], sem)

    def finish(g, idxv, rowsv, sem):
        for j in range(NDMA):
            pltpu.make_async_copy(tbl_hbm.at[idxv.at[j]],
                                  rowsv.at[pl.ds(j * IPR, IPR)], sem).wait()
        for s in range(CS):
            def body(i, accs, s=s):
                a0, a1 = accs
                r = s * L + i * UR
                for u in range(UR):
                    a0 = a0 + rowsv[r + u, pl.ds(0, 16)]
                    a1 = a1 + rowsv[r + u, pl.ds(16, 16)]
                return (a0, a1)
            z = jnp.zeros((16,), jnp.float32)
            a0, a1 = lax.fori_loop(0, L // UR, body, (z, z))
            outv[s, pl.ds(0, 16)] = a0
            outv[s, pl.ds(16, 16)] = a1
        pltpu.sync_copy(outv, out_hbm.at[pl.ds(obase + g * CS, CS)])

    start(0, idx0, rows0, sem0)
    start(1, idx1, rows1, sem1)

    def loop_body(t, carry):
        g = 2 * t
        finish(g, idx0, rows0, sem0)
        start(g + 2, idx0, rows0, sem0)
        finish(g + 1, idx1, rows1, sem1)
        start(g + 3, idx1, rows1, sem1)
        return carry

    lax.fori_loop(0, (NCH - 2) // 2, loop_body, 0)
    finish(NCH - 2, idx0, rows0, sem0)
    finish(NCH - 1, idx1, rows1, sem1)


def _embed_sum(x, weight):
    xr = x.reshape(B * RPS, IPR).astype(jnp.int32)
    f = pl.kernel(
        _emb_body,
        out_type=jax.ShapeDtypeStruct((B, D), jnp.float32),
        mesh=plsc.VectorSubcoreMesh(core_axis_name="c", subcore_axis_name="s"),
        scratch_types=[
            pltpu.VMEM((NDMA, IPR), jnp.int32),
            pltpu.VMEM((NDMA, IPR), jnp.int32),
            pltpu.VMEM((ROWS, D), jnp.float32),
            pltpu.VMEM((ROWS, D), jnp.float32),
            pltpu.VMEM((CS, D), jnp.float32),
            pltpu.SemaphoreType.DMA,
            pltpu.SemaphoreType.DMA,
        ],
        compiler_params=pltpu.CompilerParams(use_tc_tiling_on_sc=False),
    )
    return f(xr, weight)


def _mlp_body(s_ref, w1t_ref, b1_ref, w2t_ref, b2_ref, o_ref):
    h = jnp.dot(s_ref[...], w1t_ref[...], preferred_element_type=jnp.float32)
    h = jnp.maximum(h + b1_ref[...], 0.0)
    o_ref[...] = (jnp.dot(h, w2t_ref[...], preferred_element_type=jnp.float32)
                  + b2_ref[...])


def _mlp(s, W1, b1, W2, b2):
    BM = 2048
    return pl.pallas_call(
        _mlp_body,
        grid=(B // BM,),
        in_specs=[
            pl.BlockSpec((BM, D), lambda i: (i, 0)),
            pl.BlockSpec((D, HIDDEN), lambda i: (0, 0)),
            pl.BlockSpec((1, HIDDEN), lambda i: (0, 0)),
            pl.BlockSpec((HIDDEN, 1), lambda i: (0, 0)),
            pl.BlockSpec((1, 1), lambda i: (0, 0)),
        ],
        out_specs=pl.BlockSpec((BM, 1), lambda i: (i, 0)),
        out_shape=jax.ShapeDtypeStruct((B, 1), jnp.float32),
    )(s, W1.T, b1.reshape(1, HIDDEN), W2.T, b2.reshape(1, 1))


def kernel(x, weight, W1, b1, W2, b2):
    s = _embed_sum(x, weight)
    return _mlp(s, W1, b1, W2, b2)
